# speedup vs baseline: 9.1550x; 9.1550x over previous
import functools

import jax
import jax.numpy as jnp
from jax import lax
from jax.experimental import pallas as pl
from jax.experimental.pallas import tpu as pltpu
from jax.experimental.pallas import tpu_sc as plsc

P = 800
B = 8
A = 409600
L = 256
PB = 128
PPAD = 896
NBLK = PPAD // PB

BIG = 1.0e6


_NC, _NS = 2, 16
_NW = _NC * _NS
_CHUNK = A // _NW
_ROWS = _CHUNK // 128


def _seg_body(scores_hbm, ids_hbm, out_hbm, scores_v, ids_v, zeros_v, acc_sh):
    c = lax.axis_index("c")
    s = lax.axis_index("s")
    wid = c * _NS + s

    pltpu.sync_copy(scores_hbm.at[wid], scores_v)
    pltpu.sync_copy(ids_hbm.at[wid], ids_v)

    def _fill_zero(k, carry):
        zeros_v[pl.ds(k * 16, 16)] = jnp.zeros((16,), jnp.float32)
        return carry
    lax.fori_loop(0, P // 16, _fill_zero, 0)

    @pl.when(s == 0)
    def _():
        pltpu.sync_copy(zeros_v, acc_sh)

    plsc.subcore_barrier()

    def _scatter(j, carry):
        pltpu.sync_copy(scores_v.at[j], acc_sh.at[ids_v.at[j]], add=True)
        return carry
    lax.fori_loop(0, _ROWS, _scatter, 0)

    plsc.subcore_barrier()

    @pl.when(s == 0)
    def _():
        pltpu.sync_copy(acc_sh, out_hbm.at[c])


def _seg_logp(arc_scores, arc_to_path):
    scores = arc_scores.reshape(_NW, _ROWS, 128)
    ids = arc_to_path.reshape(_NW, _ROWS, 128)
    mesh = plsc.VectorSubcoreMesh(core_axis_name="c", subcore_axis_name="s")
    k = functools.partial(
        pl.kernel,
        out_type=jax.ShapeDtypeStruct((_NC, P), jnp.float32),
        mesh=mesh,
        scratch_types=[
            pltpu.VMEM((_ROWS, 128), jnp.float32),
            pltpu.VMEM((_ROWS, 128), jnp.int32),
            pltpu.VMEM((P,), jnp.float32),
            pltpu.VMEM_SHARED((P,), jnp.float32),
        ],
    )(_seg_body)
    return k(scores, ids)


def _shift_down(x, d, fill):
    pad = jnp.full((d, x.shape[1]), fill, x.dtype)
    return jnp.concatenate([pad, x[: x.shape[0] - d, :]], axis=0)


def _dp_body(hyp_ref, refsh_ref, hlen_ref, rlen_ref, out_ref):
    refsh = refsh_ref[...]
    hl = hlen_ref[...].reshape(1, PB)
    rl = rlen_ref[...].reshape(1, PB)
    jrow = lax.broadcasted_iota(jnp.int32, (L, PB), 0)

    r0 = jnp.zeros((L, PB), jnp.float32)

    def step(i, r):
        tok = hyp_ref[pl.ds(i - 1, 1), :]
        eq = refsh == tok
        a = _shift_down(r, 1, BIG) + jnp.where(eq, -1.0, 0.0)
        t = jnp.minimum(a, r + 1.0)
        for d in (1, 2, 4, 8, 16, 32, 64, 128):
            t = jnp.minimum(t, _shift_down(t, d, BIG))
        return jnp.where(i <= hl, t, r)

    r = lax.fori_loop(1, L + 1, step, r0)

    wer = jnp.sum(jnp.where(jrow == rl, r, 0.0), axis=0, keepdims=True)
    out_ref[...] = (wer + rl.astype(jnp.float32)).reshape(1, 1, PB)


def _dp_wers(hyp_T, refsh_T, hlen, rlen):
    return pl.pallas_call(
        _dp_body,
        grid=(NBLK,),
        in_specs=[
            pl.BlockSpec((L, PB), lambda b: (0, b)),
            pl.BlockSpec((L, PB), lambda b: (0, b)),
            pl.BlockSpec((1, 1, PB), lambda b: (b, 0, 0)),
            pl.BlockSpec((1, 1, PB), lambda b: (b, 0, 0)),
        ],
        out_specs=pl.BlockSpec((1, 1, PB), lambda b: (b, 0, 0)),
        out_shape=jax.ShapeDtypeStruct((NBLK, 1, PB), jnp.float32),
    )(hyp_T, refsh_T, hlen, rlen)


def _fin_body(parts_ref, wers_ref, utt_ref, out_ref):
    lp = parts_ref[0:1, :] + parts_ref[1:2, :]
    w = wers_ref[...]
    utt = utt_ref[...]
    uio = lax.broadcasted_iota(jnp.int32, (B, P), 0)
    mask = utt == uio
    mlp = jnp.where(mask, lp, -BIG)
    m = jnp.max(mlp, axis=1, keepdims=True)
    m = jnp.where(m > -0.5 * BIG, m, 0.0)
    e = jnp.where(mask, jnp.exp(lp - m), 0.0)
    ssum = jnp.sum(e, axis=1, keepdims=True)
    logz = jnp.log(ssum) + m
    prob = jnp.where(mask, jnp.exp(lp - logz), 0.0)
    loss = jnp.sum(prob * w)
    out_ref[...] = jnp.reshape(loss, (1, 1))


def _finalize(parts, wers_lin, utt_lin):
    return pl.pallas_call(
        _fin_body,
        out_shape=jax.ShapeDtypeStruct((1, 1), jnp.float32),
    )(parts, wers_lin, utt_lin)


def kernel(arc_scores, arc_to_path, path_to_utt, hyp_tokens, hyp_lens,
           ref_tokens, ref_lens):
    parts = _seg_logp(arc_scores, arc_to_path)

    ref_g = ref_tokens[path_to_utt]
    rlen_g = ref_lens[path_to_utt]
    hyp_pad = jnp.pad(hyp_tokens, ((0, PPAD - P), (0, 0)))
    ref_pad = jnp.pad(ref_g, ((0, PPAD - P), (0, 0)))
    hlen_pad = jnp.pad(hyp_lens, (0, PPAD - P))
    rlen_pad = jnp.pad(rlen_g, (0, PPAD - P))
    hyp_T = hyp_pad.T
    ref_T = ref_pad.T
    refsh_T = jnp.concatenate(
        [jnp.full((1, PPAD), -1, jnp.int32), ref_T[:-1, :]], axis=0)
    hlen3 = hlen_pad.reshape(NBLK, 1, PB)
    rlen3 = rlen_pad.reshape(NBLK, 1, PB)

    wers = _dp_wers(hyp_T, refsh_T, hlen3, rlen3)

    wers_lin = wers.reshape(1, PPAD)[:, :P]
    utt_lin = path_to_utt.reshape(1, P)
    loss = _finalize(parts, wers_lin, utt_lin)
    return loss[0, 0]
